# baseline (speedup 1.0000x reference)
import functools
import math

import jax
import jax.numpy as jnp
from jax import lax
from jax.experimental import pallas as pl
from jax.experimental.pallas import tpu as pltpu
from jax.experimental.pallas import tpu_sc as plsc

D_MODEL = 64
SCALE = math.sqrt(D_MODEL)

_NC = 2
_NS = 16
_NW = _NC * _NS

CHUNK = 512


def _make_kernel(B):
    assert B % _NW == 0
    b_per_w = B // _NW
    assert b_per_w % CHUNK == 0
    n_chunks = b_per_w // CHUNK

    mesh = plsc.VectorSubcoreMesh(core_axis_name="c", subcore_axis_name="s")

    @functools.partial(
        pl.kernel,
        mesh=mesh,
        out_type=jax.ShapeDtypeStruct((B, D_MODEL), jnp.float32),
        compiler_params=pltpu.CompilerParams(use_tc_tiling_on_sc=False),
        scratch_types=[
            pltpu.VMEM((CHUNK,), jnp.int32),
            pltpu.VMEM((CHUNK, D_MODEL), jnp.float32),
            pltpu.SemaphoreType.DMA,
        ],
    )
    def emb(x_hbm, table_hbm, out_hbm, idx_v, rows_v, sem):
        wid = lax.axis_index("s") * _NC + lax.axis_index("c")
        wbase = wid * b_per_w

        def chunk_body(g, carry):
            base = wbase + g * CHUNK
            pltpu.sync_copy(x_hbm.at[pl.ds(base, CHUNK)], idx_v)
            pltpu.async_copy(table_hbm.at[idx_v], rows_v, sem).wait()

            def row_body(r, c2):
                for c in range(D_MODEL // 16):
                    sl = pl.ds(c * 16, 16)
                    rows_v[r, sl] = rows_v[r, sl] * SCALE
                return c2

            lax.fori_loop(0, CHUNK, row_body, 0)
            pltpu.sync_copy(rows_v, out_hbm.at[pl.ds(base, CHUNK)])
            return carry

        lax.fori_loop(0, n_chunks, chunk_body, 0)

    return emb


def kernel(x, table):
    B = x.shape[0] * x.shape[1]
    flat_idx = x.reshape(B).astype(jnp.int32)
    out = _make_kernel(B)(flat_idx, table)
    return out.reshape(x.shape[0], x.shape[1], D_MODEL)


# speedup vs baseline: 1.1409x; 1.1409x over previous
import functools
import math

import jax
import jax.numpy as jnp
from jax import lax
from jax.experimental import pallas as pl
from jax.experimental.pallas import tpu as pltpu
from jax.experimental.pallas import tpu_sc as plsc

D_MODEL = 64
SCALE = math.sqrt(D_MODEL)

_NC = 2
_NS = 16
_NW = _NC * _NS

CHUNK = 320
N_BUF = 4
LEAD = 2
ROW_UNROLL = 4


def _make_kernel(B):
    assert B % _NW == 0
    b_per_w = B // _NW
    assert b_per_w % CHUNK == 0
    n_chunks = b_per_w // CHUNK
    assert n_chunks % N_BUF == 0 and n_chunks > N_BUF

    mesh = plsc.VectorSubcoreMesh(core_axis_name="c", subcore_axis_name="s")

    @functools.partial(
        pl.kernel,
        mesh=mesh,
        out_type=jax.ShapeDtypeStruct((B, D_MODEL), jnp.float32),
        compiler_params=pltpu.CompilerParams(use_tc_tiling_on_sc=False),
        scratch_types=(
            [pltpu.VMEM((b_per_w,), jnp.int32)]
            + [pltpu.VMEM((CHUNK, D_MODEL), jnp.float32) for _ in range(N_BUF)]
            + [pltpu.SemaphoreType.DMA for _ in range(2 * N_BUF)]
        ),
    )
    def emb(x_hbm, table_hbm, out_hbm, idx_all, *rest):
        rows = rest[:N_BUF]
        gsem = rest[N_BUF:2 * N_BUF]
        ssem = rest[2 * N_BUF:]

        wid = lax.axis_index("s") * _NC + lax.axis_index("c")
        wbase = wid * b_per_w

        def mk_gather(g, b):
            return pltpu.make_async_copy(
                table_hbm.at[idx_all.at[pl.ds(g * CHUNK, CHUNK)]],
                rows[b], gsem[b])

        def mk_store(g, b):
            return pltpu.make_async_copy(
                rows[b], out_hbm.at[pl.ds(wbase + g * CHUNK, CHUNK)], ssem[b])

        pltpu.sync_copy(x_hbm.at[pl.ds(wbase, b_per_w)], idx_all)
        for b in range(LEAD):
            mk_gather(b, b).start()

        def outer(i, carry):
            for b in range(N_BUF):
                g = i * N_BUF + b
                bb = (b + LEAD) % N_BUF

                @pl.when(g + LEAD < n_chunks)
                def _issue():
                    @pl.when(g >= LEAD)
                    def _drain():
                        mk_store(g - LEAD, bb).wait()
                    mk_gather(g + LEAD, bb).start()

                mk_gather(g, b).wait()

                buf = rows[b]

                def row_body(r, c):
                    for u in range(ROW_UNROLL):
                        rr = r * ROW_UNROLL + u
                        for c4 in range(D_MODEL // 16):
                            sl = pl.ds(c4 * 16, 16)
                            buf[rr, sl] = buf[rr, sl] * SCALE
                    return c

                lax.fori_loop(0, CHUNK // ROW_UNROLL, row_body, 0)
                mk_store(g, b).start()
            return carry

        lax.fori_loop(0, n_chunks // N_BUF, outer, 0)
        for g in range(n_chunks - N_BUF, n_chunks):
            mk_store(g, g % N_BUF).wait()

    return emb


def kernel(x, table):
    B = x.shape[0] * x.shape[1]
    flat_idx = x.reshape(B).astype(jnp.int32)
    out = _make_kernel(B)(flat_idx, table)
    return out.reshape(x.shape[0], x.shape[1], D_MODEL)
